# Initial kernel scaffold; baseline (speedup 1.0000x reference)
import jax
import jax.numpy as jnp
from jax.experimental import pallas as pl


def kernel(x, table, pos_encoding):
    raise NotImplementedError("write your pallas kernel here")


# speedup vs baseline: 1.1135x; 1.1135x over previous
import functools
import math

import jax
import jax.numpy as jnp
from jax import lax
from jax.experimental import pallas as pl
from jax.experimental.pallas import tpu as pltpu
from jax.experimental.pallas import tpu_sc as plsc

_NUM_CORES = 2
_NUM_SUBCORES = 16
_NUM_WORKERS = _NUM_CORES * _NUM_SUBCORES
_LANES = 16


def _emb_body(seq, n_per_w, d_emb, scale,
              x_hbm, table_hbm, pos_hbm, out_hbm, idx_v, rows_v, pos_v, sem):
    wid = lax.axis_index("s") * _NUM_CORES + lax.axis_index("c")
    base = wid * n_per_w
    pos_base = lax.rem(base, seq)

    pltpu.sync_copy(x_hbm.at[pl.ds(base, n_per_w)], idx_v)
    gather = pltpu.async_copy(table_hbm.at[idx_v], rows_v, sem)
    pltpu.sync_copy(pos_hbm.at[pl.ds(pos_base, n_per_w)], pos_v)
    gather.wait()

    n_chunks = d_emb // _LANES

    def row(r, carry):
        for c in range(n_chunks):
            sl = pl.ds(c * _LANES, _LANES)
            rows_v[r, sl] = rows_v[r, sl] * scale + pos_v[r, sl]
        return carry

    lax.fori_loop(0, n_per_w, row, 0)
    pltpu.sync_copy(rows_v, out_hbm.at[pl.ds(base, n_per_w)])


def kernel(x, table, pos_encoding):
    batch, seq = x.shape
    _, d_emb = table.shape
    n = batch * seq
    n_per_w = n // _NUM_WORKERS
    scale = math.sqrt(d_emb)

    mesh = plsc.VectorSubcoreMesh(core_axis_name="c", subcore_axis_name="s")
    body = functools.partial(_emb_body, seq, n_per_w, d_emb, scale)
    run = pl.kernel(
        body,
        mesh=mesh,
        out_type=jax.ShapeDtypeStruct((n, d_emb), jnp.float32),
        scratch_types=[
            pltpu.VMEM((n_per_w,), jnp.int32),
            pltpu.VMEM((n_per_w, d_emb), jnp.float32),
            pltpu.VMEM((n_per_w, d_emb), jnp.float32),
            pltpu.SemaphoreType.DMA,
        ],
    )
    out = run(x.reshape(n), table, pos_encoding[:seq])
    return out.reshape(batch, seq, d_emb)


# speedup vs baseline: 1.1262x; 1.0114x over previous
import functools
import math

import jax
import jax.numpy as jnp
from jax import lax
from jax.experimental import pallas as pl
from jax.experimental.pallas import tpu as pltpu
from jax.experimental.pallas import tpu_sc as plsc

_NUM_CORES = 2
_NUM_SUBCORES = 16
_NUM_WORKERS = _NUM_CORES * _NUM_SUBCORES
_LANES = 16
_CHUNKS = 4


def _emb_body(seq, n_per_w, d_emb, scale,
              x_hbm, table_hbm, pos_hbm, out_hbm,
              idx_v, rows_v, pos_v, sg0, sg1, sg2, sg3, sp, sw):
    wid = lax.axis_index("s") * _NUM_CORES + lax.axis_index("c")
    base = wid * n_per_w
    pos_base = lax.rem(base, seq)
    rows_per_chunk = n_per_w // _CHUNKS
    n_col_chunks = d_emb // _LANES
    gather_sems = (sg0, sg1, sg2, sg3)

    pltpu.sync_copy(x_hbm.at[wid], idx_v)
    gathers = [
        pltpu.async_copy(table_hbm.at[idx_v.at[c]], rows_v.at[c], gather_sems[c])
        for c in range(_CHUNKS)
    ]
    pos_cp = pltpu.async_copy(pos_hbm.at[pl.ds(pos_base, n_per_w)], pos_v, sp)
    pos_cp.wait()

    writebacks = []
    for c in range(_CHUNKS):
        gathers[c].wait()
        row_off = c * rows_per_chunk

        def row(r, carry, _c=c, _off=row_off):
            for cc in range(n_col_chunks):
                sl = pl.ds(cc * _LANES, _LANES)
                rows_v[_c, r, sl] = rows_v[_c, r, sl] * scale + pos_v[_off + r, sl]
            return carry

        lax.fori_loop(0, rows_per_chunk, row, 0)
        writebacks.append(
            pltpu.async_copy(
                rows_v.at[c], out_hbm.at[pl.ds(base + row_off, rows_per_chunk)], sw))
    for wb in writebacks:
        wb.wait()


def kernel(x, table, pos_encoding):
    batch, seq = x.shape
    _, d_emb = table.shape
    n = batch * seq
    n_per_w = n // _NUM_WORKERS
    rows_per_chunk = n_per_w // _CHUNKS
    scale = math.sqrt(d_emb)

    mesh = plsc.VectorSubcoreMesh(core_axis_name="c", subcore_axis_name="s")
    body = functools.partial(_emb_body, seq, n_per_w, d_emb, scale)
    run = pl.kernel(
        body,
        mesh=mesh,
        out_type=jax.ShapeDtypeStruct((n, d_emb), jnp.float32),
        scratch_types=[
            pltpu.VMEM((_CHUNKS, rows_per_chunk), jnp.int32),
            pltpu.VMEM((_CHUNKS, rows_per_chunk, d_emb), jnp.float32),
            pltpu.VMEM((n_per_w, d_emb), jnp.float32),
            pltpu.SemaphoreType.DMA,
            pltpu.SemaphoreType.DMA,
            pltpu.SemaphoreType.DMA,
            pltpu.SemaphoreType.DMA,
            pltpu.SemaphoreType.DMA,
            pltpu.SemaphoreType.DMA,
        ],
    )
    out = run(x.reshape(_NUM_WORKERS, _CHUNKS, rows_per_chunk), table,
              pos_encoding[:seq])
    return out.reshape(batch, seq, d_emb)


# speedup vs baseline: 1.1745x; 1.0429x over previous
import functools
import math

import jax
import jax.numpy as jnp
from jax import lax
from jax.experimental import pallas as pl
from jax.experimental.pallas import tpu as pltpu
from jax.experimental.pallas import tpu_sc as plsc

_NUM_CORES = 2
_NUM_SUBCORES = 16
_NUM_WORKERS = _NUM_CORES * _NUM_SUBCORES
_LANES = 16


def _emb_body(batch, seq, d_emb, scale,
              x_hbm, table_hbm, pos_hbm, out_hbm,
              idx_v, rows_v, pos_v, si, sp, sg0, sg1, sg2, sg3, sw):
    wid = lax.axis_index("s") * _NUM_CORES + lax.axis_index("c")
    seq_per_w = seq // _NUM_WORKERS
    seq_base = wid * seq_per_w
    n_col_chunks = d_emb // _LANES
    gather_sems = (sg0, sg1, sg2, sg3)

    idx_cps = [
        pltpu.async_copy(x_hbm.at[c, wid], idx_v.at[c], si) for c in range(batch)
    ]
    pos_cp = pltpu.async_copy(pos_hbm.at[pl.ds(seq_base, seq_per_w)], pos_v, sp)
    for cp in idx_cps:
        cp.wait()
    gathers = [
        pltpu.async_copy(table_hbm.at[idx_v.at[c]], rows_v.at[c], gather_sems[c])
        for c in range(batch)
    ]
    pos_cp.wait()

    writebacks = []
    for c in range(batch):
        gathers[c].wait()

        def row(r, carry, _c=c):
            for cc in range(n_col_chunks):
                sl = pl.ds(cc * _LANES, _LANES)
                rows_v[_c, r, sl] = rows_v[_c, r, sl] * scale + pos_v[r, sl]
            return carry

        lax.fori_loop(0, seq_per_w, row, 0)
        writebacks.append(
            pltpu.async_copy(
                rows_v.at[c], out_hbm.at[pl.ds(c * seq + seq_base, seq_per_w)], sw))
    for wb in writebacks:
        wb.wait()


def kernel(x, table, pos_encoding):
    batch, seq = x.shape
    _, d_emb = table.shape
    n = batch * seq
    seq_per_w = seq // _NUM_WORKERS
    scale = math.sqrt(d_emb)

    mesh = plsc.VectorSubcoreMesh(core_axis_name="c", subcore_axis_name="s")
    body = functools.partial(_emb_body, batch, seq, d_emb, scale)
    run = pl.kernel(
        body,
        mesh=mesh,
        out_type=jax.ShapeDtypeStruct((n, d_emb), jnp.float32),
        scratch_types=[
            pltpu.VMEM((batch, seq_per_w), jnp.int32),
            pltpu.VMEM((batch, seq_per_w, d_emb), jnp.float32),
            pltpu.VMEM((seq_per_w, d_emb), jnp.float32),
            pltpu.SemaphoreType.DMA,
            pltpu.SemaphoreType.DMA,
            pltpu.SemaphoreType.DMA,
            pltpu.SemaphoreType.DMA,
            pltpu.SemaphoreType.DMA,
            pltpu.SemaphoreType.DMA,
            pltpu.SemaphoreType.DMA,
        ],
    )
    out = run(x.reshape(batch, _NUM_WORKERS, seq_per_w), table, pos_encoding[:seq])
    return out.reshape(batch, seq, d_emb)


# speedup vs baseline: 1.1876x; 1.0112x over previous
import functools
import math

import jax
import jax.numpy as jnp
from jax import lax
from jax.experimental import pallas as pl
from jax.experimental.pallas import tpu as pltpu
from jax.experimental.pallas import tpu_sc as plsc

_NUM_CORES = 2
_NUM_SUBCORES = 16
_NUM_WORKERS = _NUM_CORES * _NUM_SUBCORES
_LANES = 16


def _emb_body(batch, seq, d_emb, scale,
              x_hbm, table_hbm, pos_hbm, out_hbm,
              idx_v, rows_v, pos_v, si, sp, sg0, sg1, sg2, sg3, sw):
    wid = lax.axis_index("s") * _NUM_CORES + lax.axis_index("c")
    seq_per_w = seq // _NUM_WORKERS
    seq_base = wid * seq_per_w
    n_col_chunks = d_emb // _LANES
    gather_sems = (sg0, sg1, sg2, sg3)

    idx_cps = [
        pltpu.async_copy(x_hbm.at[c, wid], idx_v.at[c], si) for c in range(batch)
    ]
    pos_cp = pltpu.async_copy(pos_hbm.at[pl.ds(seq_base, seq_per_w)], pos_v, sp)
    for cp in idx_cps:
        cp.wait()
    gathers = [
        pltpu.async_copy(table_hbm.at[idx_v.at[c]], rows_v.at[c], gather_sems[c])
        for c in range(batch)
    ]
    pos_cp.wait()

    writebacks = []
    for pair in range(batch // 2):
        c0, c1 = 2 * pair, 2 * pair + 1
        gathers[c0].wait()
        gathers[c1].wait()

        def row(r, carry, _c0=c0, _c1=c1):
            slices = [pl.ds(cc * _LANES, _LANES) for cc in range(n_col_chunks)]
            pos_regs = [pos_v[r, sl] for sl in slices]
            for c in (_c0, _c1):
                for cc, sl in enumerate(slices):
                    rows_v[c, r, sl] = rows_v[c, r, sl] * scale + pos_regs[cc]
            return carry

        lax.fori_loop(0, seq_per_w, row, 0)
        for c in (c0, c1):
            writebacks.append(
                pltpu.async_copy(
                    rows_v.at[c],
                    out_hbm.at[pl.ds(c * seq + seq_base, seq_per_w)], sw))
    for wb in writebacks:
        wb.wait()


def kernel(x, table, pos_encoding):
    batch, seq = x.shape
    _, d_emb = table.shape
    n = batch * seq
    seq_per_w = seq // _NUM_WORKERS
    scale = math.sqrt(d_emb)

    mesh = plsc.VectorSubcoreMesh(core_axis_name="c", subcore_axis_name="s")
    body = functools.partial(_emb_body, batch, seq, d_emb, scale)
    run = pl.kernel(
        body,
        mesh=mesh,
        out_type=jax.ShapeDtypeStruct((n, d_emb), jnp.float32),
        scratch_types=[
            pltpu.VMEM((batch, seq_per_w), jnp.int32),
            pltpu.VMEM((batch, seq_per_w, d_emb), jnp.float32),
            pltpu.VMEM((seq_per_w, d_emb), jnp.float32),
            pltpu.SemaphoreType.DMA,
            pltpu.SemaphoreType.DMA,
            pltpu.SemaphoreType.DMA,
            pltpu.SemaphoreType.DMA,
            pltpu.SemaphoreType.DMA,
            pltpu.SemaphoreType.DMA,
            pltpu.SemaphoreType.DMA,
        ],
    )
    out = run(x.reshape(batch, _NUM_WORKERS, seq_per_w), table, pos_encoding[:seq])
    return out.reshape(batch, seq, d_emb)
